# baseline (speedup 1.0000x reference)
import functools

import jax
import jax.numpy as jnp
from jax import lax
from jax.experimental import pallas as pl
from jax.experimental.pallas import tpu as pltpu
from jax.experimental.pallas import tpu_sc as plsc

_LANES = 16


def _emb_kernel(B, L, D):
    NC, NS = 2, 16
    NW = NC * NS
    rows_per_w = (B * L) // NW
    seq_per_w = rows_per_w // L
    vregs_per_seq = (L * D) // _LANES

    mesh = plsc.VectorSubcoreMesh(core_axis_name="c", subcore_axis_name="s")

    @functools.partial(
        pl.kernel,
        mesh=mesh,
        compiler_params=pltpu.CompilerParams(use_tc_tiling_on_sc=False),
        out_type=jax.ShapeDtypeStruct((B * L, D), jnp.float32),
        scratch_types=[
            pltpu.VMEM((rows_per_w,), jnp.int32),
            pltpu.VMEM((L, D), jnp.float32),
            pltpu.VMEM((L, D), jnp.float32),
            pltpu.SemaphoreType.DMA,
        ],
    )
    def k(x_hbm, tok_hbm, pos_hbm, out_hbm, idx_v, pos_v, rows_v, gsem):
        wid = lax.axis_index("s") * NC + lax.axis_index("c")
        base = wid * rows_per_w
        pltpu.sync_copy(x_hbm.at[pl.ds(base, rows_per_w)], idx_v)
        pltpu.sync_copy(pos_hbm.at[pl.ds(0, L), :], pos_v)

        def seq_body(s, carry):
            off = s * L
            pltpu.async_copy(
                tok_hbm.at[idx_v.at[pl.ds(off, L)]], rows_v, gsem
            ).wait()

            def add_row(r, c2):
                for c in range(D // _LANES):
                    sl = pl.ds(c * _LANES, _LANES)
                    rows_v[r, sl] = rows_v[r, sl] + pos_v[r, sl]
                return c2

            lax.fori_loop(0, L, add_row, 0, unroll=2)
            pltpu.sync_copy(rows_v, out_hbm.at[pl.ds(base + off, L), :])
            return carry

        lax.fori_loop(0, seq_per_w, seq_body, 0)

    return k


def kernel(x, token_table, pos_table):
    B, L = x.shape
    D = token_table.shape[1]
    xf = x.reshape(B * L).astype(jnp.int32)
    out = _emb_kernel(B, L, D)(xf, token_table, pos_table)
    return out.reshape(B, L, D)


# speedup vs baseline: 1.4473x; 1.4473x over previous
import functools

import jax
import jax.numpy as jnp
from jax import lax
from jax.experimental import pallas as pl
from jax.experimental.pallas import tpu as pltpu
from jax.experimental.pallas import tpu_sc as plsc

_LANES = 16
_NBUF = 4


def _emb_kernel(B, L, D):
    NC, NS = 2, 16
    NW = NC * NS
    seq_per_w = B // NW

    mesh = plsc.VectorSubcoreMesh(core_axis_name="c", subcore_axis_name="s")

    @functools.partial(
        pl.kernel,
        mesh=mesh,
        compiler_params=pltpu.CompilerParams(use_tc_tiling_on_sc=False),
        out_type=jax.ShapeDtypeStruct((B, L, D), jnp.float32),
        scratch_types=(
            [
                pltpu.VMEM((seq_per_w, L), jnp.int32),
                pltpu.VMEM((L, D), jnp.float32),
            ]
            + [pltpu.VMEM((L, D), jnp.float32)] * _NBUF
            + [pltpu.SemaphoreType.DMA] * (2 * _NBUF)
        ),
    )
    def k(x_hbm, tok_hbm, pos_hbm, out_hbm, idx_v, pos_v,
          r0, r1, r2, r3, g0, g1, g2, g3, s0, s1, s2, s3):
        rows = (r0, r1, r2, r3)
        gsems = (g0, g1, g2, g3)
        ssems = (s0, s1, s2, s3)
        wid = lax.axis_index("s") * NC + lax.axis_index("c")
        seq0 = wid * seq_per_w
        pltpu.sync_copy(x_hbm.at[pl.ds(seq0, seq_per_w), :], idx_v)
        pltpu.sync_copy(pos_hbm.at[pl.ds(0, L), :], pos_v)

        for b in range(_NBUF - 1):
            pltpu.async_copy(tok_hbm.at[idx_v.at[b]], rows[b], gsems[b])

        def body(g, carry):
            for b in range(_NBUF):
                s = g * _NBUF + b
                buf = rows[b]
                pltpu.make_async_copy(
                    tok_hbm.at[idx_v.at[0]], buf, gsems[b]
                ).wait()

                @plsc.parallel_loop(0, L, unroll=8)
                def _add(r):
                    for c in range(D // _LANES):
                        sl = pl.ds(c * _LANES, _LANES)
                        buf[r, sl] = buf[r, sl] + pos_v[r, sl]

                pltpu.async_copy(buf, out_hbm.at[seq0 + s], ssems[b])

                nb = (b + _NBUF - 1) % _NBUF
                ns = s + _NBUF - 1

                @pl.when(ns < seq_per_w)
                def _issue():
                    @pl.when(s >= 1)
                    def _drain():
                        pltpu.make_async_copy(
                            rows[nb], out_hbm.at[seq0], ssems[nb]
                        ).wait()

                    pltpu.async_copy(
                        tok_hbm.at[idx_v.at[ns]], rows[nb], gsems[nb]
                    )

            return carry

        lax.fori_loop(0, seq_per_w // _NBUF, body, 0)

        for b in range(_NBUF):
            pltpu.make_async_copy(
                rows[b], out_hbm.at[seq0], ssems[b]
            ).wait()

    return k


def kernel(x, token_table, pos_table):
    B, L = x.shape
    D = token_table.shape[1]
    out = _emb_kernel(B, L, D)(x.astype(jnp.int32), token_table, pos_table)
    return out
